# baseline (speedup 1.0000x reference)
import functools

import jax
import jax.numpy as jnp
from jax import lax
from jax.experimental import pallas as pl
from jax.experimental.pallas import tpu as pltpu
from jax.experimental.pallas import tpu_sc as plsc

N = 10000
E = 320000
ROW_BLK = 10112
NPAD = 10112
GRID = NPAD // ROW_BLK
NC = 2
NS = 16
NW = NC * NS
EPW = 10240
EPAD = NW * EPW
RPT = NPAD // NS
DEG_W = 16

W_CHUNK = 128
N_BATCH = 1024
N_NB = EPW // N_BATCH


@functools.cache
def _mesh():
  return plsc.VectorSubcoreMesh(
      core_axis_name="c", subcore_axis_name="s", num_cores=NC, num_subcores=NS)


@functools.cache
def _make_propagate_wide():
  n_batch = EPW // W_CHUNK
  n_phase = 2
  idx_rows = n_batch // n_phase

  @functools.partial(
      pl.kernel,
      out_type=[
          jax.ShapeDtypeStruct((NPAD, 128), jnp.float32),
          jax.ShapeDtypeStruct((NPAD, 128), jnp.float32),
      ],
      mesh=_mesh(),
      scratch_types=[
          pltpu.VMEM((idx_rows, W_CHUNK), jnp.int32),
          pltpu.VMEM((idx_rows, W_CHUNK), jnp.int32),
          pltpu.VMEM((W_CHUNK, 128), jnp.float32),
          pltpu.VMEM((W_CHUNK, 128), jnp.float32),
          pltpu.VMEM_SHARED((NPAD, 128), jnp.float32),
          pltpu.SemaphoreType.DMA,
          pltpu.SemaphoreType.DMA,
      ],
      compiler_params=pltpu.CompilerParams(use_tc_tiling_on_sc=True),
  )
  def propagate(u_hbm, edges_hbm, out0, out1, src_v, dst_v, rows0, rows1, acc,
                sem0, sem1):
    c = lax.axis_index("c")
    s = lax.axis_index("s")
    wid = c * NS + s

    def load_idx(ph):
      base = wid * n_batch + ph * idx_rows
      pltpu.sync_copy(edges_hbm.at[0, pl.ds(base, idx_rows)], src_v)
      pltpu.sync_copy(edges_hbm.at[1, pl.ds(base, idx_rows)], dst_v)

    pltpu.sync_copy(u_hbm.at[pl.ds(s * RPT, RPT)], acc.at[pl.ds(s * RPT, RPT)])
    load_idx(0)
    plsc.subcore_barrier()

    def body(i, carry):
      j0 = 2 * i
      pltpu.async_copy(u_hbm.at[src_v.at[j0 + 1]], rows1, sem1)
      pltpu.make_async_copy(u_hbm.at[src_v.at[j0]], rows0, sem0).wait()
      pltpu.sync_copy(rows0, acc.at[dst_v.at[j0]], add=True)

      @pl.when(j0 + 2 < idx_rows)
      def _():
        pltpu.async_copy(u_hbm.at[src_v.at[j0 + 2]], rows0, sem0)

      pltpu.make_async_copy(u_hbm.at[src_v.at[j0 + 1]], rows1, sem1).wait()
      pltpu.sync_copy(rows1, acc.at[dst_v.at[j0 + 1]], add=True)
      return carry

    for ph in range(n_phase):
      if ph:
        load_idx(ph)
      pltpu.async_copy(u_hbm.at[src_v.at[0]], rows0, sem0)
      lax.fori_loop(0, idx_rows // 2, body, 0)
    plsc.subcore_barrier()

    @pl.when(c == 0)
    def _():
      pltpu.sync_copy(acc.at[pl.ds(s * RPT, RPT)], out0.at[pl.ds(s * RPT, RPT)])

    @pl.when(c == 1)
    def _():
      pltpu.sync_copy(acc.at[pl.ds(s * RPT, RPT)], out1.at[pl.ds(s * RPT, RPT)])

  return propagate


@functools.cache
def _make_propagate_narrow(D):
  @functools.partial(
      pl.kernel,
      out_type=[
          jax.ShapeDtypeStruct((NPAD, D), jnp.float32),
          jax.ShapeDtypeStruct((NPAD, D), jnp.float32),
      ],
      mesh=_mesh(),
      scratch_types=[
          pltpu.VMEM((N_NB, N_BATCH), jnp.int32),
          pltpu.VMEM((N_NB, N_BATCH), jnp.int32),
          pltpu.VMEM((N_BATCH, D), jnp.float32),
          pltpu.VMEM((N_BATCH, D), jnp.float32),
          pltpu.VMEM_SHARED((NPAD, D), jnp.float32),
          pltpu.VMEM_SHARED((NPAD, D), jnp.float32),
          pltpu.SemaphoreType.DMA,
          pltpu.SemaphoreType.DMA,
      ],
      compiler_params=pltpu.CompilerParams(use_tc_tiling_on_sc=False),
  )
  def propagate(u_hbm, edges_hbm, out0, out1, src_v, dst_v, rows0, rows1, acc,
                u_sp, sem0, sem1):
    c = lax.axis_index("c")
    s = lax.axis_index("s")
    wid = c * NS + s
    pltpu.sync_copy(u_hbm.at[pl.ds(s * RPT, RPT)], acc.at[pl.ds(s * RPT, RPT)])
    pltpu.sync_copy(u_hbm.at[pl.ds(s * RPT, RPT)], u_sp.at[pl.ds(s * RPT, RPT)])
    pltpu.sync_copy(edges_hbm.at[0, pl.ds(wid * N_NB, N_NB)], src_v)
    pltpu.sync_copy(edges_hbm.at[1, pl.ds(wid * N_NB, N_NB)], dst_v)
    plsc.subcore_barrier()

    def body(i, carry):
      j0 = 2 * i
      pltpu.async_copy(u_sp.at[src_v.at[j0 + 1]], rows1, sem1)
      pltpu.make_async_copy(u_sp.at[src_v.at[j0]], rows0, sem0).wait()
      pltpu.sync_copy(rows0, acc.at[dst_v.at[j0]], add=True)

      @pl.when(j0 + 2 < N_NB)
      def _():
        pltpu.async_copy(u_sp.at[src_v.at[j0 + 2]], rows0, sem0)

      pltpu.make_async_copy(u_sp.at[src_v.at[j0 + 1]], rows1, sem1).wait()
      pltpu.sync_copy(rows1, acc.at[dst_v.at[j0 + 1]], add=True)
      return carry

    pltpu.async_copy(u_sp.at[src_v.at[0]], rows0, sem0)
    lax.fori_loop(0, N_NB // 2, body, 0)
    plsc.subcore_barrier()

    @pl.when(c == 0)
    def _():
      pltpu.sync_copy(acc.at[pl.ds(s * RPT, RPT)], out0.at[pl.ds(s * RPT, RPT)])

    @pl.when(c == 1)
    def _():
      pltpu.sync_copy(acc.at[pl.ds(s * RPT, RPT)], out1.at[pl.ds(s * RPT, RPT)])

  return propagate


@functools.cache
def _make_degree():
  @functools.partial(
      pl.kernel,
      out_type=[
          jax.ShapeDtypeStruct((NPAD, DEG_W), jnp.float32),
          jax.ShapeDtypeStruct((NPAD, DEG_W), jnp.float32),
      ],
      mesh=_mesh(),
      scratch_types=[
          pltpu.VMEM((N_NB, N_BATCH), jnp.int32),
          pltpu.VMEM((N_BATCH, DEG_W), jnp.float32),
          pltpu.VMEM_SHARED((NPAD, DEG_W), jnp.float32),
          pltpu.SemaphoreType.DMA,
      ],
      compiler_params=pltpu.CompilerParams(use_tc_tiling_on_sc=False),
  )
  def degree(edges_hbm, ones_hbm, zeros_hbm, out0, out1, dst_v, ones_v, acc,
             sem):
    c = lax.axis_index("c")
    s = lax.axis_index("s")
    wid = c * NS + s
    pltpu.sync_copy(zeros_hbm.at[pl.ds(s * RPT, RPT)],
                    acc.at[pl.ds(s * RPT, RPT)])
    pltpu.sync_copy(ones_hbm, ones_v)
    pltpu.sync_copy(edges_hbm.at[1, pl.ds(wid * N_NB, N_NB)], dst_v)
    plsc.subcore_barrier()

    def body(j, carry):
      pltpu.sync_copy(ones_v, acc.at[dst_v.at[j]], add=True)
      return carry

    lax.fori_loop(0, N_NB, body, 0)
    plsc.subcore_barrier()

    @pl.when(c == 0)
    def _():
      pltpu.sync_copy(acc.at[pl.ds(s * RPT, RPT)], out0.at[pl.ds(s * RPT, RPT)])

    @pl.when(c == 1)
    def _():
      pltpu.sync_copy(acc.at[pl.ds(s * RPT, RPT)], out1.at[pl.ds(s * RPT, RPT)])

  return degree


def _dinv_col(d0_ref, d1_ref):
  return lax.rsqrt(d0_ref[:, :1] + d1_ref[:, :1] + 1.0)


def _mm_scale_body(x_ref, w_ref, d0_ref, d1_ref, u_ref):
  dinv = _dinv_col(d0_ref, d1_ref)
  u_ref[...] = (
      jnp.dot(x_ref[...], w_ref[...], preferred_element_type=jnp.float32)
      * dinv)


def _combine_mm_body(p0_ref, p1_ref, u_ref, d0_ref, d1_ref, w_ref, b_ref,
                     o_ref):
  dinv = _dinv_col(d0_ref, d1_ref)
  h = jnp.maximum(
      dinv * (p0_ref[...] + p1_ref[...] - u_ref[...]) + b_ref[...], 0.0)
  o_ref[...] = (
      jnp.dot(h, w_ref[...], preferred_element_type=jnp.float32) * dinv)


def _head_body(p0_ref, p1_ref, u_ref, d0_ref, d1_ref, b3_ref, w1_ref, c1_ref,
               w2_ref, c2_ref, o_ref):
  dinv = _dinv_col(d0_ref, d1_ref)
  h3 = jnp.maximum(
      dinv * (p0_ref[...] + p1_ref[...] - u_ref[...]) + b3_ref[...], 0.0)
  h4 = jnp.maximum(
      jnp.dot(h3, w1_ref[...], preferred_element_type=jnp.float32)
      + c1_ref[...], 0.0)
  o_ref[...] = (
      jnp.dot(h4, w2_ref[...], preferred_element_type=jnp.float32)
      + c2_ref[...])


def _rows(shape):
  return pl.BlockSpec((ROW_BLK, shape), lambda i: (i, 0))


def _full(shape):
  return pl.BlockSpec(shape, lambda i: (0,) * len(shape))


def _mm_scale(xp, w, d0, d1):
  fo = w.shape[1]
  return pl.pallas_call(
      _mm_scale_body,
      grid=(GRID,),
      in_specs=[
          _rows(xp.shape[1]), _full(w.shape), _rows(DEG_W), _rows(DEG_W)
      ],
      out_specs=_rows(fo),
      out_shape=jax.ShapeDtypeStruct((NPAD, fo), jnp.float32),
  )(xp, w, d0, d1)


def _combine_mm(p0, p1, u, d0, d1, w, b):
  fi = u.shape[1]
  fo = w.shape[1]
  return pl.pallas_call(
      _combine_mm_body,
      grid=(GRID,),
      in_specs=[
          _rows(fi), _rows(fi), _rows(fi), _rows(DEG_W), _rows(DEG_W),
          _full(w.shape), _full(b.shape)
      ],
      out_specs=_rows(fo),
      out_shape=jax.ShapeDtypeStruct((NPAD, fo), jnp.float32),
  )(p0, p1, u, d0, d1, w, b)


def _head(p0, p1, u, d0, d1, b3, w1, c1, w2, c2):
  fo = w2.shape[1]
  return pl.pallas_call(
      _head_body,
      grid=(GRID,),
      in_specs=[
          _rows(16), _rows(16), _rows(16), _rows(DEG_W), _rows(DEG_W),
          _full(b3.shape), _full(w1.shape), _full(c1.shape), _full(w2.shape),
          _full(c2.shape)
      ],
      out_specs=_rows(fo),
      out_shape=jax.ShapeDtypeStruct((N, fo), jnp.float32),
  )(p0, p1, u, d0, d1, b3, w1, c1, w2, c2)


def kernel(x, edge_index, W1, b1, W2, b2, W3, b3, L1W, L1b, L2W, L2b):
  ei = edge_index.astype(jnp.int32)
  n_pad_e = EPAD - E
  pad_iota = jnp.arange(n_pad_e, dtype=jnp.int32)
  pad_pair = jnp.stack([pad_iota % N, N + pad_iota % (NPAD - N)])
  ei_pad = jnp.concatenate([ei, pad_pair], axis=1)
  e128 = ei_pad.reshape(2, EPAD // W_CHUNK, W_CHUNK)
  e1024 = ei_pad.reshape(2, EPAD // N_BATCH, N_BATCH)
  ones = jnp.ones((N_BATCH, DEG_W), jnp.float32)
  zeros = jnp.zeros((NPAD, DEG_W), jnp.float32)

  d0, d1 = _make_degree()(e1024, ones, zeros)

  u1 = _mm_scale(x, W1, d0, d1)
  p0, p1 = _make_propagate_wide()(u1, e128)
  u2 = _combine_mm(p0, p1, u1, d0, d1, W2, b1.reshape(1, -1))
  q0, q1 = _make_propagate_narrow(32)(u2, e1024)
  u3 = _combine_mm(q0, q1, u2, d0, d1, W3, b2.reshape(1, -1))
  r0, r1 = _make_propagate_narrow(16)(u3, e1024)
  return _head(r0, r1, u3, d0, d1, b3.reshape(1, -1), L1W, L1b.reshape(1, -1),
               L2W, L2b.reshape(1, -1))


# speedup vs baseline: 1.0149x; 1.0149x over previous
import functools

import jax
import jax.numpy as jnp
from jax import lax
from jax.experimental import pallas as pl
from jax.experimental.pallas import tpu as pltpu
from jax.experimental.pallas import tpu_sc as plsc

N = 10000
E = 320000
ROW_BLK = 2528
NPAD = 10112
GRID = NPAD // ROW_BLK
NC = 2
NS = 16
NW = NC * NS
EPW = 10240
EPAD = NW * EPW
RPT = NPAD // NS
DEG_W = 16

W_CHUNK = 128
N_BATCH = 1024
N_NB = EPW // N_BATCH


@functools.cache
def _mesh():
  return plsc.VectorSubcoreMesh(
      core_axis_name="c", subcore_axis_name="s", num_cores=NC, num_subcores=NS)


@functools.cache
def _make_propagate_wide():
  n_batch = EPW // W_CHUNK
  n_phase = 2
  idx_rows = n_batch // n_phase

  @functools.partial(
      pl.kernel,
      out_type=[
          jax.ShapeDtypeStruct((NPAD, 128), jnp.float32),
          jax.ShapeDtypeStruct((NPAD, 128), jnp.float32),
      ],
      mesh=_mesh(),
      scratch_types=[
          pltpu.VMEM((idx_rows, W_CHUNK), jnp.int32),
          pltpu.VMEM((idx_rows, W_CHUNK), jnp.int32),
          pltpu.VMEM((W_CHUNK, 128), jnp.float32),
          pltpu.VMEM((W_CHUNK, 128), jnp.float32),
          pltpu.VMEM_SHARED((NPAD, 128), jnp.float32),
          pltpu.SemaphoreType.DMA,
          pltpu.SemaphoreType.DMA,
      ],
      compiler_params=pltpu.CompilerParams(use_tc_tiling_on_sc=True),
  )
  def propagate(u_hbm, edges_hbm, out0, out1, src_v, dst_v, rows0, rows1, acc,
                sem0, sem1):
    c = lax.axis_index("c")
    s = lax.axis_index("s")
    wid = c * NS + s

    def load_idx(ph):
      base = wid * n_batch + ph * idx_rows
      pltpu.sync_copy(edges_hbm.at[0, pl.ds(base, idx_rows)], src_v)
      pltpu.sync_copy(edges_hbm.at[1, pl.ds(base, idx_rows)], dst_v)

    pltpu.sync_copy(u_hbm.at[pl.ds(s * RPT, RPT)], acc.at[pl.ds(s * RPT, RPT)])
    load_idx(0)
    plsc.subcore_barrier()

    def body(i, carry):
      j0 = 2 * i
      pltpu.async_copy(u_hbm.at[src_v.at[j0 + 1]], rows1, sem1)
      pltpu.make_async_copy(u_hbm.at[src_v.at[j0]], rows0, sem0).wait()
      pltpu.sync_copy(rows0, acc.at[dst_v.at[j0]], add=True)

      @pl.when(j0 + 2 < idx_rows)
      def _():
        pltpu.async_copy(u_hbm.at[src_v.at[j0 + 2]], rows0, sem0)

      pltpu.make_async_copy(u_hbm.at[src_v.at[j0 + 1]], rows1, sem1).wait()
      pltpu.sync_copy(rows1, acc.at[dst_v.at[j0 + 1]], add=True)
      return carry

    for ph in range(n_phase):
      if ph:
        load_idx(ph)
      pltpu.async_copy(u_hbm.at[src_v.at[0]], rows0, sem0)
      lax.fori_loop(0, idx_rows // 2, body, 0)
    plsc.subcore_barrier()

    @pl.when(c == 0)
    def _():
      pltpu.sync_copy(acc.at[pl.ds(s * RPT, RPT)], out0.at[pl.ds(s * RPT, RPT)])

    @pl.when(c == 1)
    def _():
      pltpu.sync_copy(acc.at[pl.ds(s * RPT, RPT)], out1.at[pl.ds(s * RPT, RPT)])

  return propagate


@functools.cache
def _make_propagate_narrow(D):
  @functools.partial(
      pl.kernel,
      out_type=[
          jax.ShapeDtypeStruct((NPAD, D), jnp.float32),
          jax.ShapeDtypeStruct((NPAD, D), jnp.float32),
      ],
      mesh=_mesh(),
      scratch_types=[
          pltpu.VMEM((N_NB, N_BATCH), jnp.int32),
          pltpu.VMEM((N_NB, N_BATCH), jnp.int32),
          pltpu.VMEM((N_BATCH, D), jnp.float32),
          pltpu.VMEM((N_BATCH, D), jnp.float32),
          pltpu.VMEM_SHARED((NPAD, D), jnp.float32),
          pltpu.VMEM_SHARED((NPAD, D), jnp.float32),
          pltpu.SemaphoreType.DMA,
          pltpu.SemaphoreType.DMA,
      ],
      compiler_params=pltpu.CompilerParams(use_tc_tiling_on_sc=False),
  )
  def propagate(u_hbm, edges_hbm, out0, out1, src_v, dst_v, rows0, rows1, acc,
                u_sp, sem0, sem1):
    c = lax.axis_index("c")
    s = lax.axis_index("s")
    wid = c * NS + s
    pltpu.sync_copy(u_hbm.at[pl.ds(s * RPT, RPT)], acc.at[pl.ds(s * RPT, RPT)])
    pltpu.sync_copy(u_hbm.at[pl.ds(s * RPT, RPT)], u_sp.at[pl.ds(s * RPT, RPT)])
    pltpu.sync_copy(edges_hbm.at[0, pl.ds(wid * N_NB, N_NB)], src_v)
    pltpu.sync_copy(edges_hbm.at[1, pl.ds(wid * N_NB, N_NB)], dst_v)
    plsc.subcore_barrier()

    def body(i, carry):
      j0 = 2 * i
      pltpu.async_copy(u_sp.at[src_v.at[j0 + 1]], rows1, sem1)
      pltpu.make_async_copy(u_sp.at[src_v.at[j0]], rows0, sem0).wait()
      pltpu.sync_copy(rows0, acc.at[dst_v.at[j0]], add=True)

      @pl.when(j0 + 2 < N_NB)
      def _():
        pltpu.async_copy(u_sp.at[src_v.at[j0 + 2]], rows0, sem0)

      pltpu.make_async_copy(u_sp.at[src_v.at[j0 + 1]], rows1, sem1).wait()
      pltpu.sync_copy(rows1, acc.at[dst_v.at[j0 + 1]], add=True)
      return carry

    pltpu.async_copy(u_sp.at[src_v.at[0]], rows0, sem0)
    lax.fori_loop(0, N_NB // 2, body, 0)
    plsc.subcore_barrier()

    @pl.when(c == 0)
    def _():
      pltpu.sync_copy(acc.at[pl.ds(s * RPT, RPT)], out0.at[pl.ds(s * RPT, RPT)])

    @pl.when(c == 1)
    def _():
      pltpu.sync_copy(acc.at[pl.ds(s * RPT, RPT)], out1.at[pl.ds(s * RPT, RPT)])

  return propagate


@functools.cache
def _make_degree():
  @functools.partial(
      pl.kernel,
      out_type=[
          jax.ShapeDtypeStruct((NPAD, DEG_W), jnp.float32),
          jax.ShapeDtypeStruct((NPAD, DEG_W), jnp.float32),
      ],
      mesh=_mesh(),
      scratch_types=[
          pltpu.VMEM((N_NB, N_BATCH), jnp.int32),
          pltpu.VMEM((N_BATCH, DEG_W), jnp.float32),
          pltpu.VMEM_SHARED((NPAD, DEG_W), jnp.float32),
          pltpu.SemaphoreType.DMA,
      ],
      compiler_params=pltpu.CompilerParams(use_tc_tiling_on_sc=False),
  )
  def degree(edges_hbm, ones_hbm, zeros_hbm, out0, out1, dst_v, ones_v, acc,
             sem):
    c = lax.axis_index("c")
    s = lax.axis_index("s")
    wid = c * NS + s
    pltpu.sync_copy(zeros_hbm.at[pl.ds(s * RPT, RPT)],
                    acc.at[pl.ds(s * RPT, RPT)])
    pltpu.sync_copy(ones_hbm, ones_v)
    pltpu.sync_copy(edges_hbm.at[1, pl.ds(wid * N_NB, N_NB)], dst_v)
    plsc.subcore_barrier()

    def body(j, carry):
      pltpu.sync_copy(ones_v, acc.at[dst_v.at[j]], add=True)
      return carry

    lax.fori_loop(0, N_NB, body, 0)
    plsc.subcore_barrier()

    @pl.when(c == 0)
    def _():
      pltpu.sync_copy(acc.at[pl.ds(s * RPT, RPT)], out0.at[pl.ds(s * RPT, RPT)])

    @pl.when(c == 1)
    def _():
      pltpu.sync_copy(acc.at[pl.ds(s * RPT, RPT)], out1.at[pl.ds(s * RPT, RPT)])

  return degree


def _dinv_col(d0_ref, d1_ref):
  return lax.rsqrt(d0_ref[:, :1] + d1_ref[:, :1] + 1.0)


def _mm_scale_body(x_ref, w_ref, d0_ref, d1_ref, u_ref):
  dinv = _dinv_col(d0_ref, d1_ref)
  u_ref[...] = (
      jnp.dot(x_ref[...], w_ref[...], preferred_element_type=jnp.float32)
      * dinv)


def _combine_mm_body(p0_ref, p1_ref, u_ref, d0_ref, d1_ref, w_ref, b_ref,
                     o_ref):
  dinv = _dinv_col(d0_ref, d1_ref)
  h = jnp.maximum(
      dinv * (p0_ref[...] + p1_ref[...] - u_ref[...]) + b_ref[...], 0.0)
  o_ref[...] = (
      jnp.dot(h, w_ref[...], preferred_element_type=jnp.float32) * dinv)


def _head_body(p0_ref, p1_ref, u_ref, d0_ref, d1_ref, b3_ref, w1_ref, c1_ref,
               w2_ref, c2_ref, o_ref):
  dinv = _dinv_col(d0_ref, d1_ref)
  h3 = jnp.maximum(
      dinv * (p0_ref[...] + p1_ref[...] - u_ref[...]) + b3_ref[...], 0.0)
  h4 = jnp.maximum(
      jnp.dot(h3, w1_ref[...], preferred_element_type=jnp.float32)
      + c1_ref[...], 0.0)
  o_ref[...] = (
      jnp.dot(h4, w2_ref[...], preferred_element_type=jnp.float32)
      + c2_ref[...])


def _rows(shape):
  return pl.BlockSpec((ROW_BLK, shape), lambda i: (i, 0))


def _full(shape):
  return pl.BlockSpec(shape, lambda i: (0,) * len(shape))


def _mm_scale(xp, w, d0, d1):
  fo = w.shape[1]
  return pl.pallas_call(
      _mm_scale_body,
      grid=(GRID,),
      in_specs=[
          _rows(xp.shape[1]), _full(w.shape), _rows(DEG_W), _rows(DEG_W)
      ],
      out_specs=_rows(fo),
      out_shape=jax.ShapeDtypeStruct((NPAD, fo), jnp.float32),
  )(xp, w, d0, d1)


def _combine_mm(p0, p1, u, d0, d1, w, b):
  fi = u.shape[1]
  fo = w.shape[1]
  return pl.pallas_call(
      _combine_mm_body,
      grid=(GRID,),
      in_specs=[
          _rows(fi), _rows(fi), _rows(fi), _rows(DEG_W), _rows(DEG_W),
          _full(w.shape), _full(b.shape)
      ],
      out_specs=_rows(fo),
      out_shape=jax.ShapeDtypeStruct((NPAD, fo), jnp.float32),
  )(p0, p1, u, d0, d1, w, b)


def _head(p0, p1, u, d0, d1, b3, w1, c1, w2, c2):
  fo = w2.shape[1]
  return pl.pallas_call(
      _head_body,
      grid=(GRID,),
      in_specs=[
          _rows(16), _rows(16), _rows(16), _rows(DEG_W), _rows(DEG_W),
          _full(b3.shape), _full(w1.shape), _full(c1.shape), _full(w2.shape),
          _full(c2.shape)
      ],
      out_specs=_rows(fo),
      out_shape=jax.ShapeDtypeStruct((N, fo), jnp.float32),
  )(p0, p1, u, d0, d1, b3, w1, c1, w2, c2)


def kernel(x, edge_index, W1, b1, W2, b2, W3, b3, L1W, L1b, L2W, L2b):
  ei = edge_index.astype(jnp.int32)
  n_pad_e = EPAD - E
  pad_iota = jnp.arange(n_pad_e, dtype=jnp.int32)
  pad_pair = jnp.stack([pad_iota % N, N + pad_iota % (NPAD - N)])
  ei_pad = jnp.concatenate([ei, pad_pair], axis=1)
  e128 = ei_pad.reshape(2, EPAD // W_CHUNK, W_CHUNK)
  e1024 = ei_pad.reshape(2, EPAD // N_BATCH, N_BATCH)
  ones = jnp.ones((N_BATCH, DEG_W), jnp.float32)
  zeros = jnp.zeros((NPAD, DEG_W), jnp.float32)

  d0, d1 = _make_degree()(e1024, ones, zeros)

  u1 = _mm_scale(x, W1, d0, d1)
  p0, p1 = _make_propagate_wide()(u1, e128)
  u2 = _combine_mm(p0, p1, u1, d0, d1, W2, b1.reshape(1, -1))
  q0, q1 = _make_propagate_narrow(32)(u2, e1024)
  u3 = _combine_mm(q0, q1, u2, d0, d1, W3, b2.reshape(1, -1))
  r0, r1 = _make_propagate_narrow(16)(u3, e1024)
  return _head(r0, r1, u3, d0, d1, b3.reshape(1, -1), L1W, L1b.reshape(1, -1),
               L2W, L2b.reshape(1, -1))
